# Initial kernel scaffold; baseline (speedup 1.0000x reference)
import jax
import jax.numpy as jnp
from jax.experimental import pallas as pl


def kernel(x_cat, x_cont, median, factors):
    raise NotImplementedError("write your pallas kernel here")


# speedup vs baseline: 2.0512x; 2.0512x over previous
import functools

import jax
import jax.numpy as jnp
from jax import lax
from jax.experimental import pallas as pl
from jax.experimental.pallas import tpu as pltpu
from jax.experimental.pallas import tpu_sc as plsc

B = 16384
NCAT = 26
CATSZ = 100
NCONT = 13
D = NCAT * CATSZ + NCONT


_RBLK = 256


def _dense_body(xc_ref, med_ref, fac_ref, out_ref):
    x = xc_ref[...]
    xs = fac_ref[...] * (x - med_ref[...])
    y = xs / jnp.sqrt(1.0 + (xs * (1.0 / 3.0)) ** 2)
    out_ref[...] = jnp.zeros((_RBLK, D), jnp.float32)
    out_ref[:, NCAT * CATSZ:D] = y


_dense_call = pl.pallas_call(
    _dense_body,
    grid=(B // _RBLK,),
    in_specs=[
        pl.BlockSpec((_RBLK, NCONT), lambda i: (i, 0)),
        pl.BlockSpec((1, NCONT), lambda i: (0, 0)),
        pl.BlockSpec((1, NCONT), lambda i: (0, 0)),
    ],
    out_specs=pl.BlockSpec((_RBLK, D), lambda i: (i, 0)),
    out_shape=jax.ShapeDtypeStruct((B, D), jnp.float32),
)


_NW = 32
_RPW = B // _NW
_WPW = _RPW * NCAT
_IDXROWS = _WPW // 128

_sc_mesh = plsc.VectorSubcoreMesh(core_axis_name="c", subcore_axis_name="s")


@functools.partial(
    pl.kernel,
    mesh=_sc_mesh,
    scratch_types=[
        pltpu.VMEM((_WPW,), jnp.int32),
        pltpu.VMEM((_IDXROWS, 128), jnp.int32),
        pltpu.VMEM((128,), jnp.float32),
        pltpu.SemaphoreType.DMA,
    ],
)
def _sc_scatter(out_hbm, cat_hbm, cat_v, idx_v, ones_v, sem):
    wid = lax.axis_index("s") * 2 + lax.axis_index("c")
    row0 = wid * _RPW

    for i in range(NCAT):
        pltpu.sync_copy(
            cat_hbm.at[pl.ds(i * B + row0, _RPW)],
            cat_v.at[pl.ds(i * _RPW, _RPW)],
        )

    for b in range(8):
        ones_v[pl.ds(b * 16, 16)] = jnp.full((16,), 1.0, jnp.float32)

    lane_d = lax.iota(jnp.int32, 16) * D

    def fill(g, carry):
        i = g >> 2
        r0 = (g & 3) * 128
        for b in range(8):
            cat = cat_v[pl.ds(i * _RPW + r0 + b * 16, 16)]
            scalar = (row0 + r0 + b * 16) * D + i * CATSZ
            idx = lane_d + cat + scalar
            idx_v[g, pl.ds(b * 16, 16)] = idx
        return carry

    lax.fori_loop(0, _IDXROWS, fill, 0)

    def fire(g, carry):
        copies = [
            pltpu.async_copy(ones_v, out_hbm.at[idx_v.at[g * 8 + b]], sem)
            for b in range(8)
        ]
        for c in copies:
            c.wait()
        return carry

    lax.fori_loop(0, _IDXROWS // 8, fire, 0)


def kernel(x_cat, x_cont, median, factors):
    cat_flat = x_cat.astype(jnp.int32).T.reshape(-1)
    dense = _dense_call(
        x_cont.astype(jnp.float32),
        median.astype(jnp.float32).reshape(1, NCONT),
        factors.astype(jnp.float32).reshape(1, NCONT),
    )
    out_ref = jax.new_ref(dense.reshape(-1))
    _sc_scatter(out_ref, cat_flat)
    return out_ref[...].reshape(B, D)
